# baseline (speedup 1.0000x reference)
import functools

import jax
import jax.numpy as jnp
from jax.experimental import pallas as pl
from jax.experimental.pallas import tpu as pltpu

_VMEM_LIMIT = 48 * 1024 * 1024
_EPS = 1e-5
_PHASES = ((0, 0), (0, 1), (1, 0), (1, 1))


def _l0_bn_relu_kernel(x_ref, w_ref, g_ref, b_ref, o_ref, *, m_rows, cout, eps):
    y = jnp.dot(x_ref[...], w_ref[...], preferred_element_type=jnp.float32)
    s = jnp.sum(y, axis=0, keepdims=True)
    ss = jnp.sum(y * y, axis=0, keepdims=True)
    mean = 0.0
    msq = 0.0
    for t in range(16):
        mean = mean + s[:, t * cout:(t + 1) * cout]
        msq = msq + ss[:, t * cout:(t + 1) * cout]
    inv_m = 1.0 / m_rows
    mean = mean * inv_m
    var = msq * inv_m - mean * mean
    inv = jax.lax.rsqrt(var + eps)
    scale = g_ref[...] * inv
    shift = b_ref[...] - mean * scale
    scale16 = jnp.concatenate([scale] * 16, axis=1)
    shift16 = jnp.concatenate([shift] * 16, axis=1)
    o_ref[...] = jnp.maximum(y * scale16 + shift16, 0.0).astype(jnp.bfloat16)


def _zmat_kernel(x_ref, w_ref, o_ref):
    o_ref[...] = jnp.dot(x_ref[...], w_ref[...],
                         preferred_element_type=jnp.float32)


def _overlap_bn_relu_kernel(z_ref, g_ref, b_ref, o_ref, *, m_total, eps, hh, ww):
    ys = []
    s = jnp.zeros_like(g_ref[...])
    ss = jnp.zeros_like(g_ref[...])
    for ph, pw in _PHASES:
        y = None
        for kh in (1 - ph, 3 - ph):
            sh = (ph - kh + 3) // 2
            for kw in (1 - pw, 3 - pw):
                sw = (pw - kw + 3) // 2
                t = z_ref[kh * 4 + kw, :, sh:sh + hh, sw:sw + ww, :]
                y = t if y is None else y + t
        ys.append(y)
        s = s + jnp.sum(y, axis=(0, 1, 2))[None, :]
        ss = ss + jnp.sum(y * y, axis=(0, 1, 2))[None, :]
    inv_m = 1.0 / m_total
    mean = s * inv_m
    var = ss * inv_m - mean * mean
    inv = jax.lax.rsqrt(var + eps)
    scale = g_ref[...] * inv
    shift = b_ref[...] - mean * scale
    for i in range(4):
        o_ref[i] = jnp.maximum(ys[i] * scale + shift, 0.0).astype(jnp.bfloat16)


def _phase_tanh_kernel(p_ref, w_ref, b_ref, o_ref):
    for i in range(4):
        y = jnp.dot(p_ref[i], w_ref[i], preferred_element_type=jnp.float32)
        o_ref[i] = jnp.tanh(y + b_ref[...])


def _l0_layer(x2d, w_mat, gamma, beta, n, cout):
    m, k = x2d.shape
    kern = functools.partial(_l0_bn_relu_kernel, m_rows=float(16 * n),
                             cout=cout, eps=_EPS)
    vmem = pl.BlockSpec(memory_space=pltpu.MemorySpace.VMEM)
    o = pl.pallas_call(
        kern,
        out_shape=jax.ShapeDtypeStruct((m, 16 * cout), jnp.bfloat16),
        in_specs=[vmem] * 4,
        out_specs=vmem,
        compiler_params=pltpu.CompilerParams(vmem_limit_bytes=_VMEM_LIMIT),
    )(x2d, w_mat, gamma[None, :], beta[None, :])
    return o.reshape(n, 4, 4, cout)


def _zmat(xflat, w_native):
    m, k = xflat.shape
    l = w_native.shape[1]
    return pl.pallas_call(
        _zmat_kernel,
        out_shape=jax.ShapeDtypeStruct((m, l), jnp.float32),
        grid=(2,),
        in_specs=[pl.BlockSpec((m, k), lambda i: (0, 0)),
                  pl.BlockSpec((k, l // 2), lambda i: (0, i))],
        out_specs=pl.BlockSpec((m, l // 2), lambda i: (0, i)),
        compiler_params=pltpu.CompilerParams(
            dimension_semantics=("parallel",),
            vmem_limit_bytes=_VMEM_LIMIT),
    )(xflat, w_native)


def _overlap_bn_relu(zp, gamma, beta, m_total):
    _, n, hp, wp, cout = zp.shape
    hh, ww = hp - 2, wp - 2
    nc = 2 if cout >= 256 else 1
    tc = cout // nc
    kern = functools.partial(_overlap_bn_relu_kernel, m_total=float(m_total),
                             eps=_EPS, hh=hh, ww=ww)
    return pl.pallas_call(
        kern,
        out_shape=jax.ShapeDtypeStruct((4, n, hh, ww, cout), jnp.bfloat16),
        grid=(nc,),
        in_specs=[pl.BlockSpec((16, n, hp, wp, tc), lambda i: (0, 0, 0, 0, i)),
                  pl.BlockSpec((1, tc), lambda i: (0, i)),
                  pl.BlockSpec((1, tc), lambda i: (0, i))],
        out_specs=pl.BlockSpec((4, n, hh, ww, tc), lambda i: (0, 0, 0, 0, i)),
        compiler_params=pltpu.CompilerParams(
            dimension_semantics=("parallel",),
            vmem_limit_bytes=_VMEM_LIMIT),
    )(zp, gamma[None, :], beta[None, :])


def _phase_tanh_layer(patches, w_stk, bias, tm=512):
    _, m, k = patches.shape
    cout = w_stk.shape[-1]
    tm = min(tm, m)
    return pl.pallas_call(
        _phase_tanh_kernel,
        out_shape=jax.ShapeDtypeStruct((4, m, cout), jnp.float32),
        grid=(m // tm,),
        in_specs=[pl.BlockSpec((4, tm, k), lambda i: (0, i, 0)),
                  pl.BlockSpec((4, k, cout), lambda i: (0, 0, 0)),
                  pl.BlockSpec((1, cout), lambda i: (0, 0))],
        out_specs=pl.BlockSpec((4, tm, cout), lambda i: (0, i, 0)),
        compiler_params=pltpu.CompilerParams(
            dimension_semantics=("parallel",),
            vmem_limit_bytes=_VMEM_LIMIT),
    )(patches, w_stk, bias[None, :])


def _convt_bn_relu_layer(h, wt, gamma, beta):
    n, hh, ww, cin = h.shape
    cout = wt.shape[1]
    w_native = wt.astype(jnp.bfloat16).reshape(cin, 16 * cout)
    z = _zmat(h.reshape(n * hh * ww, cin), w_native)
    zt = z.reshape(n, hh, ww, cout, 16).transpose(4, 0, 1, 2, 3)
    zp = jnp.pad(zt, ((0, 0), (0, 0), (1, 1), (1, 1), (0, 0)))
    o4 = _overlap_bn_relu(zp, gamma, beta, 4 * n * hh * ww)
    return _interleave(o4.reshape(4, n * hh * ww, cout), n, hh, ww, cout)


def _phase_patches(x):
    n, h, w, c = x.shape
    xp = jnp.pad(x, ((0, 0), (1, 1), (1, 1), (0, 0)))
    stk = []
    for ph, pw in _PHASES:
        cols = [xp[:, ph + dy:ph + dy + h, pw + dx:pw + dx + w, :]
                for dy in (0, 1) for dx in (0, 1)]
        stk.append(jnp.concatenate(cols, axis=-1).reshape(n * h * w, 4 * c))
    return jnp.stack(stk)


def _phase_weights(wt):
    cin, cout = wt.shape[0], wt.shape[1]
    w16 = wt.astype(jnp.bfloat16).transpose(0, 2, 3, 1).reshape(cin, 16, cout)
    mats = []
    for ph, pw in _PHASES:
        taps = [(3 - ph - 2 * dy) * 4 + (3 - pw - 2 * dx)
                for dy in (0, 1) for dx in (0, 1)]
        sub = jnp.stack([w16[:, t, :] for t in taps], axis=0)
        mats.append(sub.reshape(4 * cin, cout))
    return jnp.stack(mats)


def _interleave(o4, n, h, w, c):
    o = o4.reshape(2, 2, n, h, w, c).transpose(2, 3, 0, 4, 1, 5)
    return o.reshape(n, 2 * h, 2 * w, c)


def _l0_weight(w0):
    cin, cout = w0.shape[0], w0.shape[1]
    m = w0.astype(jnp.bfloat16).transpose(0, 2, 3, 1).reshape(cin, 16 * cout)
    return jnp.pad(m, ((0, 128 - cin), (0, 0)))


def kernel(x, W0, g0, b0, W1, g1, b1, W2, g2, b2, W3, g3, b3, fW, fb):
    n = x.shape[0]
    x2d = jnp.pad(x.reshape(n, 100), ((0, 0), (0, 28))).astype(jnp.bfloat16)

    h = _l0_layer(x2d, _l0_weight(W0), g0, b0, n, 1024)

    for wt, g, b in ((W1, g1, b1), (W2, g2, b2), (W3, g3, b3)):
        h = _convt_bn_relu_layer(h, wt, g, b)

    nn, hh, ww, cc = h.shape
    fw_stk = jnp.pad(_phase_weights(fW), ((0, 0), (0, 0), (0, 125)))
    fb_p = jnp.pad(fb, (0, 125))
    o4 = _phase_tanh_layer(_phase_patches(h), fw_stk, fb_p)
    o = _interleave(o4, nn, hh, ww, 128)[..., :3]
    return jnp.transpose(o, (0, 3, 1, 2))


# speedup vs baseline: 1.9665x; 1.9665x over previous
import functools

import jax
import jax.numpy as jnp
from jax.experimental import pallas as pl
from jax.experimental.pallas import tpu as pltpu

_VMEM_LIMIT = 48 * 1024 * 1024
_EPS = 1e-5
_PHASES = ((0, 0), (0, 1), (1, 0), (1, 1))


def _l0_bn_relu_kernel(x_ref, w_ref, g_ref, b_ref, o_ref, *, m_rows, cout, eps):
    y = jnp.dot(x_ref[...], w_ref[...], preferred_element_type=jnp.float32)
    s = jnp.sum(y, axis=0, keepdims=True)
    ss = jnp.sum(y * y, axis=0, keepdims=True)
    mean = 0.0
    msq = 0.0
    for t in range(16):
        mean = mean + s[:, t * cout:(t + 1) * cout]
        msq = msq + ss[:, t * cout:(t + 1) * cout]
    inv_m = 1.0 / m_rows
    mean = mean * inv_m
    var = msq * inv_m - mean * mean
    inv = jax.lax.rsqrt(var + eps)
    scale = g_ref[...] * inv
    shift = b_ref[...] - mean * scale
    scale16 = jnp.concatenate([scale] * 16, axis=1)
    shift16 = jnp.concatenate([shift] * 16, axis=1)
    o_ref[...] = jnp.maximum(y * scale16 + shift16, 0.0).astype(jnp.bfloat16)


def _zmat_kernel(x_ref, w_ref, o_ref):
    o_ref[...] = jnp.dot(x_ref[...], w_ref[...],
                         preferred_element_type=jnp.float32)


def _overlap_bn_relu_kernel(z_ref, g_ref, b_ref, o_ref, *, m_total, eps,
                            hh, ww, cout):
    ys = []
    s = jnp.zeros_like(g_ref[...])
    ss = jnp.zeros_like(g_ref[...])
    for ph, pw in _PHASES:
        y = None
        for kh in (1 - ph, 3 - ph):
            sh = (ph - kh + 3) // 2
            for kw in (1 - pw, 3 - pw):
                sw = (pw - kw + 3) // 2
                lane = (kh * 4 + kw) * cout
                t = z_ref[:, sh:sh + hh, sw:sw + ww, lane:lane + cout]
                y = t if y is None else y + t
        ys.append(y)
        s = s + jnp.sum(y, axis=(0, 1, 2))[None, :]
        ss = ss + jnp.sum(y * y, axis=(0, 1, 2))[None, :]
    inv_m = 1.0 / m_total
    mean = s * inv_m
    var = ss * inv_m - mean * mean
    inv = jax.lax.rsqrt(var + eps)
    scale = g_ref[...] * inv
    shift = b_ref[...] - mean * scale
    for i in range(4):
        o_ref[i] = jnp.maximum(ys[i] * scale + shift, 0.0).astype(jnp.bfloat16)


def _phase_tanh_kernel(p_ref, w_ref, b_ref, o_ref):
    for i in range(4):
        y = jnp.dot(p_ref[i], w_ref[i], preferred_element_type=jnp.float32)
        o_ref[i] = jnp.tanh(y + b_ref[...])


def _l0_layer(x2d, w_mat, gamma, beta, n, cout):
    m, k = x2d.shape
    kern = functools.partial(_l0_bn_relu_kernel, m_rows=float(16 * n),
                             cout=cout, eps=_EPS)
    vmem = pl.BlockSpec(memory_space=pltpu.MemorySpace.VMEM)
    o = pl.pallas_call(
        kern,
        out_shape=jax.ShapeDtypeStruct((m, 16 * cout), jnp.bfloat16),
        in_specs=[vmem] * 4,
        out_specs=vmem,
        compiler_params=pltpu.CompilerParams(vmem_limit_bytes=_VMEM_LIMIT),
    )(x2d, w_mat, gamma[None, :], beta[None, :])
    return o.reshape(n, 4, 4, cout)


def _zmat(xflat, w_native):
    m, k = xflat.shape
    l = w_native.shape[1]
    return pl.pallas_call(
        _zmat_kernel,
        out_shape=jax.ShapeDtypeStruct((m, l), jnp.float32),
        grid=(2,),
        in_specs=[pl.BlockSpec((m, k), lambda i: (0, 0)),
                  pl.BlockSpec((k, l // 2), lambda i: (0, i))],
        out_specs=pl.BlockSpec((m, l // 2), lambda i: (0, i)),
        compiler_params=pltpu.CompilerParams(
            dimension_semantics=("parallel",),
            vmem_limit_bytes=_VMEM_LIMIT),
    )(xflat, w_native)


def _overlap_bn_relu(zp, gamma, beta, m_total, cout):
    n, hp, wp, _ = zp.shape
    hh, ww = hp - 2, wp - 2
    kern = functools.partial(_overlap_bn_relu_kernel, m_total=float(m_total),
                             eps=_EPS, hh=hh, ww=ww, cout=cout)
    vmem = pl.BlockSpec(memory_space=pltpu.MemorySpace.VMEM)
    return pl.pallas_call(
        kern,
        out_shape=jax.ShapeDtypeStruct((4, n, hh, ww, cout), jnp.bfloat16),
        in_specs=[vmem] * 3,
        out_specs=vmem,
        compiler_params=pltpu.CompilerParams(vmem_limit_bytes=_VMEM_LIMIT),
    )(zp, gamma[None, :], beta[None, :])


def _phase_tanh_layer(patches, w_stk, bias, tm=512):
    _, m, k = patches.shape
    cout = w_stk.shape[-1]
    tm = min(tm, m)
    return pl.pallas_call(
        _phase_tanh_kernel,
        out_shape=jax.ShapeDtypeStruct((4, m, cout), jnp.float32),
        grid=(m // tm,),
        in_specs=[pl.BlockSpec((4, tm, k), lambda i: (0, i, 0)),
                  pl.BlockSpec((4, k, cout), lambda i: (0, 0, 0)),
                  pl.BlockSpec((1, cout), lambda i: (0, 0))],
        out_specs=pl.BlockSpec((4, tm, cout), lambda i: (0, i, 0)),
        compiler_params=pltpu.CompilerParams(
            dimension_semantics=("parallel",),
            vmem_limit_bytes=_VMEM_LIMIT),
    )(patches, w_stk, bias[None, :])


def _convt_bn_relu_layer(h, wt, gamma, beta):
    n, hh, ww, cin = h.shape
    cout = wt.shape[1]
    w_tap = wt.astype(jnp.bfloat16).transpose(0, 2, 3, 1).reshape(cin, 16 * cout)
    z = _zmat(h.reshape(n * hh * ww, cin), w_tap)
    zp = jnp.pad(z.reshape(n, hh, ww, 16 * cout),
                 ((0, 0), (1, 1), (1, 1), (0, 0)))
    o4 = _overlap_bn_relu(zp, gamma, beta, 4 * n * hh * ww, cout)
    return _interleave(o4.reshape(4, n * hh * ww, cout), n, hh, ww, cout)


def _phase_patches(x):
    n, h, w, c = x.shape
    xp = jnp.pad(x, ((0, 0), (1, 1), (1, 1), (0, 0)))
    stk = []
    for ph, pw in _PHASES:
        cols = [xp[:, ph + dy:ph + dy + h, pw + dx:pw + dx + w, :]
                for dy in (0, 1) for dx in (0, 1)]
        stk.append(jnp.concatenate(cols, axis=-1).reshape(n * h * w, 4 * c))
    return jnp.stack(stk)


def _phase_weights(wt):
    cin, cout = wt.shape[0], wt.shape[1]
    w16 = wt.astype(jnp.bfloat16).transpose(0, 2, 3, 1).reshape(cin, 16, cout)
    mats = []
    for ph, pw in _PHASES:
        taps = [(3 - ph - 2 * dy) * 4 + (3 - pw - 2 * dx)
                for dy in (0, 1) for dx in (0, 1)]
        sub = jnp.stack([w16[:, t, :] for t in taps], axis=0)
        mats.append(sub.reshape(4 * cin, cout))
    return jnp.stack(mats)


def _interleave(o4, n, h, w, c):
    o = o4.reshape(2, 2, n, h, w, c).transpose(2, 3, 0, 4, 1, 5)
    return o.reshape(n, 2 * h, 2 * w, c)


def _l0_weight(w0):
    cin, cout = w0.shape[0], w0.shape[1]
    m = w0.astype(jnp.bfloat16).transpose(0, 2, 3, 1).reshape(cin, 16 * cout)
    return jnp.pad(m, ((0, 128 - cin), (0, 0)))


def kernel(x, W0, g0, b0, W1, g1, b1, W2, g2, b2, W3, g3, b3, fW, fb):
    n = x.shape[0]
    x2d = jnp.pad(x.reshape(n, 100), ((0, 0), (0, 28))).astype(jnp.bfloat16)

    h = _l0_layer(x2d, _l0_weight(W0), g0, b0, n, 1024)

    for wt, g, b in ((W1, g1, b1), (W2, g2, b2), (W3, g3, b3)):
        h = _convt_bn_relu_layer(h, wt, g, b)

    nn, hh, ww, cc = h.shape
    fw_stk = jnp.pad(_phase_weights(fW), ((0, 0), (0, 0), (0, 125)))
    fb_p = jnp.pad(fb, (0, 125))
    o4 = _phase_tanh_layer(_phase_patches(h), fw_stk, fb_p)
    o = _interleave(o4, nn, hh, ww, 128)[..., :3]
    return jnp.transpose(o, (0, 3, 1, 2))


# speedup vs baseline: 1.9669x; 1.0002x over previous
import functools

import jax
import jax.numpy as jnp
from jax.experimental import pallas as pl
from jax.experimental.pallas import tpu as pltpu

_VMEM_LIMIT = 48 * 1024 * 1024
_EPS = 1e-5
_PHASES = ((0, 0), (0, 1), (1, 0), (1, 1))


def _l0_bn_relu_kernel(x_ref, w_ref, g_ref, b_ref, o_ref, *, m_rows, cout, eps):
    y = jnp.dot(x_ref[...], w_ref[...], preferred_element_type=jnp.float32)
    s = jnp.sum(y, axis=0, keepdims=True)
    ss = jnp.sum(y * y, axis=0, keepdims=True)
    mean = 0.0
    msq = 0.0
    for t in range(16):
        mean = mean + s[:, t * cout:(t + 1) * cout]
        msq = msq + ss[:, t * cout:(t + 1) * cout]
    inv_m = 1.0 / m_rows
    mean = mean * inv_m
    var = msq * inv_m - mean * mean
    inv = jax.lax.rsqrt(var + eps)
    scale = g_ref[...] * inv
    shift = b_ref[...] - mean * scale
    scale16 = jnp.concatenate([scale] * 16, axis=1)
    shift16 = jnp.concatenate([shift] * 16, axis=1)
    o_ref[...] = jnp.maximum(y * scale16 + shift16, 0.0).astype(jnp.bfloat16)


def _zmat_kernel(x_ref, w_ref, o_ref):
    o_ref[...] = jnp.dot(x_ref[...], w_ref[...],
                         preferred_element_type=jnp.float32)


def _overlap_bn_relu_kernel(z_ref, g_ref, b_ref, o_ref, *, m_total, eps,
                            hh, ww, cout):
    ys = []
    s = jnp.zeros_like(g_ref[...])
    ss = jnp.zeros_like(g_ref[...])
    for ph, pw in _PHASES:
        y = None
        for kh in (1 - ph, 3 - ph):
            sh = (ph - kh + 3) // 2
            for kw in (1 - pw, 3 - pw):
                sw = (pw - kw + 3) // 2
                lane = (kh * 4 + kw) * cout
                t = z_ref[:, sh:sh + hh, sw:sw + ww, lane:lane + cout]
                y = t if y is None else y + t
        ys.append(y)
        s = s + jnp.sum(y, axis=(0, 1, 2))[None, :]
        ss = ss + jnp.sum(y * y, axis=(0, 1, 2))[None, :]
    inv_m = 1.0 / m_total
    mean = s * inv_m
    var = ss * inv_m - mean * mean
    inv = jax.lax.rsqrt(var + eps)
    scale = g_ref[...] * inv
    shift = b_ref[...] - mean * scale
    for i in range(4):
        o_ref[i] = jnp.maximum(ys[i] * scale + shift, 0.0).astype(jnp.bfloat16)


def _phase_tanh_kernel(p_ref, w_ref, b_ref, o_ref):
    for i in range(4):
        y = jnp.dot(p_ref[i], w_ref[i], preferred_element_type=jnp.float32)
        o_ref[i] = jnp.tanh(y + b_ref[...])


def _l0_layer(x2d, w_mat, gamma, beta, n, cout):
    m, k = x2d.shape
    kern = functools.partial(_l0_bn_relu_kernel, m_rows=float(16 * n),
                             cout=cout, eps=_EPS)
    vmem = pl.BlockSpec(memory_space=pltpu.MemorySpace.VMEM)
    o = pl.pallas_call(
        kern,
        out_shape=jax.ShapeDtypeStruct((m, 16 * cout), jnp.bfloat16),
        in_specs=[vmem] * 4,
        out_specs=vmem,
        compiler_params=pltpu.CompilerParams(vmem_limit_bytes=_VMEM_LIMIT),
    )(x2d, w_mat, gamma[None, :], beta[None, :])
    return o.reshape(n, 4, 4, cout)


def _zmat(xflat, w_native):
    m, k = xflat.shape
    l = w_native.shape[1]
    return pl.pallas_call(
        _zmat_kernel,
        out_shape=jax.ShapeDtypeStruct((m, l), jnp.float32),
        grid=(2,),
        in_specs=[pl.BlockSpec((m, k), lambda i: (0, 0)),
                  pl.BlockSpec((k, l // 2), lambda i: (0, i))],
        out_specs=pl.BlockSpec((m, l // 2), lambda i: (0, i)),
        compiler_params=pltpu.CompilerParams(
            dimension_semantics=("parallel",),
            vmem_limit_bytes=_VMEM_LIMIT),
    )(xflat, w_native)


def _overlap_bn_relu(zp, gamma, beta, m_total, cout):
    n, hp, wp, _ = zp.shape
    hh, ww = hp - 2, wp - 2
    kern = functools.partial(_overlap_bn_relu_kernel, m_total=float(m_total),
                             eps=_EPS, hh=hh, ww=ww, cout=cout)
    vmem = pl.BlockSpec(memory_space=pltpu.MemorySpace.VMEM)
    return pl.pallas_call(
        kern,
        out_shape=jax.ShapeDtypeStruct((4, n, hh, ww, cout), jnp.bfloat16),
        in_specs=[vmem] * 3,
        out_specs=vmem,
        compiler_params=pltpu.CompilerParams(vmem_limit_bytes=_VMEM_LIMIT),
    )(zp, gamma[None, :], beta[None, :])


def _phase_tanh_layer(patches, w_stk, bias, tm=512):
    _, m, k = patches.shape
    cout = w_stk.shape[-1]
    tm = min(tm, m)
    return pl.pallas_call(
        _phase_tanh_kernel,
        out_shape=jax.ShapeDtypeStruct((4, m, cout), jnp.float32),
        grid=(m // tm,),
        in_specs=[pl.BlockSpec((4, tm, k), lambda i: (0, i, 0)),
                  pl.BlockSpec((4, k, cout), lambda i: (0, 0, 0)),
                  pl.BlockSpec((1, cout), lambda i: (0, 0))],
        out_specs=pl.BlockSpec((4, tm, cout), lambda i: (0, i, 0)),
        compiler_params=pltpu.CompilerParams(
            dimension_semantics=("parallel",),
            vmem_limit_bytes=_VMEM_LIMIT),
    )(patches, w_stk, bias[None, :])


def _convt_bn_relu_layer(h, wt, gamma, beta):
    n, hh, ww, cin = h.shape
    cout = wt.shape[1]
    w_tap = wt.transpose(0, 2, 3, 1).reshape(cin, 16 * cout).astype(jnp.bfloat16)
    z = _zmat(h.reshape(n * hh * ww, cin), w_tap)
    zp = jnp.pad(z.reshape(n, hh, ww, 16 * cout),
                 ((0, 0), (1, 1), (1, 1), (0, 0)))
    o4 = _overlap_bn_relu(zp, gamma, beta, 4 * n * hh * ww, cout)
    return _interleave(o4.reshape(4, n * hh * ww, cout), n, hh, ww, cout)


def _phase_patches(x):
    n, h, w, c = x.shape
    xp = jnp.pad(x, ((0, 0), (1, 1), (1, 1), (0, 0)))
    stk = []
    for ph, pw in _PHASES:
        cols = [xp[:, ph + dy:ph + dy + h, pw + dx:pw + dx + w, :]
                for dy in (0, 1) for dx in (0, 1)]
        stk.append(jnp.concatenate(cols, axis=-1).reshape(n * h * w, 4 * c))
    return jnp.stack(stk)


def _phase_weights(wt):
    cin, cout = wt.shape[0], wt.shape[1]
    w16 = wt.astype(jnp.bfloat16).transpose(0, 2, 3, 1).reshape(cin, 16, cout)
    mats = []
    for ph, pw in _PHASES:
        taps = [(3 - ph - 2 * dy) * 4 + (3 - pw - 2 * dx)
                for dy in (0, 1) for dx in (0, 1)]
        sub = jnp.stack([w16[:, t, :] for t in taps], axis=0)
        mats.append(sub.reshape(4 * cin, cout))
    return jnp.stack(mats)


def _interleave(o4, n, h, w, c):
    o = o4.reshape(2, 2, n, h, w, c).transpose(2, 3, 0, 4, 1, 5)
    return o.reshape(n, 2 * h, 2 * w, c)


def _l0_weight(w0):
    cin, cout = w0.shape[0], w0.shape[1]
    m = w0.astype(jnp.bfloat16).transpose(0, 2, 3, 1).reshape(cin, 16 * cout)
    return jnp.pad(m, ((0, 128 - cin), (0, 0)))


def kernel(x, W0, g0, b0, W1, g1, b1, W2, g2, b2, W3, g3, b3, fW, fb):
    n = x.shape[0]
    x2d = jnp.pad(x.reshape(n, 100), ((0, 0), (0, 28))).astype(jnp.bfloat16)

    h = _l0_layer(x2d, _l0_weight(W0), g0, b0, n, 1024)

    for wt, g, b in ((W1, g1, b1), (W2, g2, b2), (W3, g3, b3)):
        h = _convt_bn_relu_layer(h, wt, g, b)

    nn, hh, ww, cc = h.shape
    fw_stk = jnp.pad(_phase_weights(fW), ((0, 0), (0, 0), (0, 125)))
    fb_p = jnp.pad(fb, (0, 125))
    o4 = _phase_tanh_layer(_phase_patches(h), fw_stk, fb_p)
    o = _interleave(o4, nn, hh, ww, 128)[..., :3]
    return jnp.transpose(o, (0, 3, 1, 2))
